# baseline (speedup 1.0000x reference)
import functools

import jax
import jax.numpy as jnp
from jax import lax
from jax.experimental import pallas as pl
from jax.experimental.pallas import tpu as pltpu
from jax.experimental.pallas import tpu_sc as plsc

NUM_USERS = 1000000
NUM_ITEMS = 100000
EMB = 64
SHW = 32
B = 16384

NC = 2
NS = 16
NW = NC * NS
PW = B // NW
CH = 128
NCH = PW // CH
L = 16


def _body(u2d_r, i2d_r, ush_r, ish_r, uemb_r, ubias_r, iemb_r, ibias_r,
          suw_r, siw_r, mean_r, out_r,
          uidx, iidx, ush_v, ish_v, ue_v, ie_v, siw_v, suw_v, bu_v, bi_v,
          outb, mean_v, sem):
    wid = lax.axis_index("s") * NC + lax.axis_index("c")
    base = wid * PW

    pltpu.sync_copy(u2d_r.at[pl.ds(wid * NCH, NCH)], uidx)
    pltpu.sync_copy(i2d_r.at[pl.ds(wid * NCH, NCH)], iidx)
    pltpu.sync_copy(ush_r.at[pl.ds(base, PW)], ush_v)
    pltpu.sync_copy(ish_r.at[pl.ds(base, PW)], ish_v)
    pltpu.sync_copy(mean_r, mean_v)
    mv = mean_v[...]
    iota = lax.iota(jnp.int32, L)
    zcol = jnp.zeros((L,), jnp.int32)

    for j in range(NCH):
        cps = [
            pltpu.async_copy(uemb_r.at[uidx.at[j]], ue_v, sem),
            pltpu.async_copy(iemb_r.at[iidx.at[j]], ie_v, sem),
            pltpu.async_copy(siw_r.at[iidx.at[j]], siw_v, sem),
            pltpu.async_copy(suw_r.at[uidx.at[j]], suw_v, sem),
            pltpu.async_copy(ubias_r.at[uidx.at[j]], bu_v, sem),
            pltpu.async_copy(ibias_r.at[iidx.at[j]], bi_v, sem),
        ]
        for cp in cps:
            cp.wait()

        def group(g, carry, j=j):
            res = bu_v[pl.ds(g * L, L)] + bi_v[pl.ds(g * L, L)] + mv
            for k in range(L):
                r = g * L + k
                br = j * CH + r
                acc = ue_v[r, pl.ds(0, L)] * ie_v[r, pl.ds(0, L)]
                for t in range(1, EMB // L):
                    acc += (ue_v[r, pl.ds(t * L, L)]
                            * ie_v[r, pl.ds(t * L, L)])
                for t in range(SHW // L):
                    acc += (ush_v[br, pl.ds(t * L, L)]
                            * siw_v[r, pl.ds(t * L, L)])
                    acc += (ish_v[br, pl.ds(t * L, L)]
                            * suw_v[r, pl.ds(t * L, L)])
                res += jnp.where(iota == k, jnp.sum(acc), 0.0)
            outb[pl.ds(j * CH + g * L, L)] = res
            return carry

        lax.fori_loop(0, CH // L, group, 0)

    pltpu.sync_copy(outb, out_r.at[pl.ds(base, PW)])


@functools.partial(jax.jit, static_argnames=())
def kernel(u_id, i_id, UserShadow, ItemShadow, user_emb_w, user_bias_w,
           item_emb_w, item_bias_w, shadow_u_w, shadow_i_w, mean):
    u2d = u_id.astype(jnp.int32).reshape(B // CH, CH)
    i2d = i_id.astype(jnp.int32).reshape(B // CH, CH)
    mean16 = jnp.broadcast_to(mean.astype(jnp.float32), (L,))
    ub1 = user_bias_w.reshape(NUM_USERS)
    ib1 = item_bias_w.reshape(NUM_ITEMS)

    f32 = jnp.float32
    run = pl.kernel(
        _body,
        out_type=jax.ShapeDtypeStruct((B,), f32),
        mesh=plsc.VectorSubcoreMesh(core_axis_name="c", subcore_axis_name="s"),
        compiler_params=pltpu.CompilerParams(
            needs_layout_passes=False, use_tc_tiling_on_sc=False),
        scratch_types=[
            pltpu.VMEM((NCH, CH), jnp.int32),
            pltpu.VMEM((NCH, CH), jnp.int32),
            pltpu.VMEM((PW, SHW), f32),
            pltpu.VMEM((PW, SHW), f32),
            pltpu.VMEM((CH, EMB), f32),
            pltpu.VMEM((CH, EMB), f32),
            pltpu.VMEM((CH, SHW), f32),
            pltpu.VMEM((CH, SHW), f32),
            pltpu.VMEM((CH,), f32),
            pltpu.VMEM((CH,), f32),
            pltpu.VMEM((PW,), f32),
            pltpu.VMEM((L,), f32),
            pltpu.SemaphoreType.DMA,
        ],
    )
    return run(u2d, i2d, UserShadow, ItemShadow, user_emb_w, ub1,
               item_emb_w, ib1, shadow_u_w, shadow_i_w, mean16)
